# baseline (speedup 1.0000x reference)
import jax
import jax.numpy as jnp
from jax.experimental import pallas as pl
from jax.experimental.pallas import tpu as pltpu
from jax.experimental.pallas import tpu_sc as plsc

_EPS = 1e-5

_NC = 2
_NS = 16
_NW = _NC * _NS
_L = 16

_HW = 1024
_ROWS = 64 * 384
_TR = _ROWS // 8
_TRPW = _TR // _NW
_CT = 3
_NCHUNK = _TRPW // _CT


def _ln(x, gamma, beta):
    mean = jnp.mean(x, axis=-1, keepdims=True)
    var = jnp.mean((x - mean) ** 2, axis=-1, keepdims=True)
    return (x - mean) * jax.lax.rsqrt(var + _EPS) * gamma + beta


def _scale_body(tv_ref, tt_ref, vt_ref, tg_ref, tb_ref, vg_ref, vb_ref,
                scale_ref):
    B = tv_ref.shape[0]
    V = tt_ref.shape[0]
    idx = tv_ref[:]
    iota = jax.lax.broadcasted_iota(jnp.int32, (B, V), 1)
    oh_t = (iota == idx[:, 0:1]).astype(jnp.float32)
    oh_v = (iota == idx[:, 1:2]).astype(jnp.float32)
    temb = jnp.dot(oh_t, tt_ref[:], preferred_element_type=jnp.float32,
                   precision=jax.lax.Precision.HIGHEST)
    vemb = jnp.dot(oh_v, vt_ref[:], preferred_element_type=jnp.float32,
                   precision=jax.lax.Precision.HIGHEST)
    tln = _ln(temb, tg_ref[:], tb_ref[:])
    vln = _ln(vemb, vg_ref[:], vb_ref[:])
    scale_ref[:] = tln * vln


def _mul_chunk(in_buf, out_buf, scale_vmem, chunk_idx):
    def sr_body(sr, _):
        for t in range(_CT):
            sidx = jnp.full((_L,), chunk_idx * (_CT * 8) + t * 8 + sr,
                            dtype=jnp.int32)
            svec = plsc.load_gather(scale_vmem, [sidx])
            for tc in range(8):
                for v in range(128 // _L):
                    sl = pl.ds(v * _L, _L)
                    out_buf[t, tc, sr, sl] = in_buf[t, tc, sr, sl] * svec
        return 0

    jax.lax.fori_loop(0, 8, sr_body, 0)


def _sc_mul_body(scale_hbm, ft_hbm, out_hbm, in0, in1, ob0, ob1, scale_vmem,
                 gsem0, gsem1, ssem0, ssem1):
    wid = jax.lax.axis_index("s") * _NC + jax.lax.axis_index("c")
    base_tr = wid * _TRPW
    base_row = wid * (_TRPW * 8)

    ins = (in0, in1)
    obs = (ob0, ob1)
    gsems = (gsem0, gsem1)
    ssems = (ssem0, ssem1)

    pltpu.sync_copy(
        scale_hbm.at[pl.ds(pl.multiple_of(base_row, _TRPW * 8), _TRPW * 8)],
        scale_vmem)

    def trs(i):
        return pl.ds(base_tr + i * _CT, _CT)

    def gather(i, s):
        return pltpu.make_async_copy(ft_hbm.at[trs(i)], ins[s], gsems[s])

    def scatter(i, s):
        return pltpu.make_async_copy(obs[s], out_hbm.at[trs(i)], ssems[s])

    gather(0, 0).start()
    gather(1, 1).start()

    def pair_body(g, _):
        for s in range(2):
            i = g * 2 + s
            gather(i, s).wait()

            @pl.when(i >= 2)
            def _():
                scatter(i - 2, s).wait()

            _mul_chunk(ins[s], obs[s], scale_vmem, i)
            scatter(i, s).start()

            @pl.when(i + 2 < _NCHUNK)
            def _():
                gather(i + 2, s).start()
        return 0

    jax.lax.fori_loop(0, _NCHUNK // 2, pair_body, 0)

    scatter(_NCHUNK - 2, 0).wait()
    scatter(_NCHUNK - 1, 1).wait()


def kernel(ft, taskvar, task_table, var_table, task_gamma, task_beta,
           var_gamma, var_beta):
    B, C, H, W = ft.shape

    scale = pl.pallas_call(
        _scale_body,
        out_shape=jax.ShapeDtypeStruct((B, C), jnp.float32),
    )(taskvar, task_table, var_table,
      task_gamma.reshape(1, C), task_beta.reshape(1, C),
      var_gamma.reshape(1, C), var_beta.reshape(1, C))

    ft4 = ft.reshape(_TR, 8, 8, 128).transpose(0, 2, 1, 3)

    mesh = plsc.VectorSubcoreMesh(core_axis_name="c", subcore_axis_name="s")
    sc_mul = pl.kernel(
        _sc_mul_body,
        out_type=jax.ShapeDtypeStruct((_TR, 8, 8, 128), jnp.float32),
        mesh=mesh,
        compiler_params=pltpu.CompilerParams(needs_layout_passes=False),
        scratch_types=[
            pltpu.VMEM((_CT, 8, 8, 128), jnp.float32),
            pltpu.VMEM((_CT, 8, 8, 128), jnp.float32),
            pltpu.VMEM((_CT, 8, 8, 128), jnp.float32),
            pltpu.VMEM((_CT, 8, 8, 128), jnp.float32),
            pltpu.VMEM((_TRPW * 8,), jnp.float32),
            pltpu.SemaphoreType.DMA,
            pltpu.SemaphoreType.DMA,
            pltpu.SemaphoreType.DMA,
            pltpu.SemaphoreType.DMA,
        ],
    )
    out4 = sc_mul(scale.reshape(_ROWS), ft4)
    out = out4.transpose(0, 2, 1, 3).reshape(B, C, H, W)
    return out


# speedup vs baseline: 4.5003x; 4.5003x over previous
import jax
import jax.numpy as jnp
from jax.experimental import pallas as pl
from jax.experimental.pallas import tpu as pltpu

_EPS = 1e-5
_BB = 8


def _ln(x, gamma, beta):
    mean = jnp.mean(x, axis=-1, keepdims=True)
    var = jnp.mean((x - mean) ** 2, axis=-1, keepdims=True)
    return (x - mean) * jax.lax.rsqrt(var + _EPS) * gamma + beta


def _body(tv_ref, ft_ref, tt_ref, vt_ref, tg_ref, tb_ref, vg_ref, vb_ref,
          out_ref, scale_ref):
    g = pl.program_id(0)

    @pl.when(g == 0)
    def _():
        B = scale_ref.shape[0]
        V = tt_ref.shape[0]
        idx = tv_ref[:]
        iota = jax.lax.broadcasted_iota(jnp.int32, (B, V), 1)
        oh_t = (iota == idx[:, 0:1]).astype(jnp.float32)
        oh_v = (iota == idx[:, 1:2]).astype(jnp.float32)
        temb = jnp.dot(oh_t, tt_ref[:], preferred_element_type=jnp.float32,
                       precision=jax.lax.Precision.HIGHEST)
        vemb = jnp.dot(oh_v, vt_ref[:], preferred_element_type=jnp.float32,
                       precision=jax.lax.Precision.HIGHEST)
        tln = _ln(temb, tg_ref[:], tb_ref[:])
        vln = _ln(vemb, vg_ref[:], vb_ref[:])
        scale_ref[:] = tln * vln

    s = scale_ref[pl.ds(g * _BB, _BB), :]
    out_ref[:] = ft_ref[:] * s[:, :, None]


def kernel(ft, taskvar, task_table, var_table, task_gamma, task_beta,
           var_gamma, var_beta):
    B, C, H, W = ft.shape
    HW = H * W
    V = task_table.shape[0]
    ft3 = ft.reshape(B, C, HW)
    out3 = pl.pallas_call(
        _body,
        grid=(B // _BB,),
        in_specs=[
            pl.BlockSpec((B, 2), lambda g: (0, 0)),
            pl.BlockSpec((_BB, C, HW), lambda g: (g, 0, 0)),
            pl.BlockSpec((V, C), lambda g: (0, 0)),
            pl.BlockSpec((V, C), lambda g: (0, 0)),
            pl.BlockSpec((1, C), lambda g: (0, 0)),
            pl.BlockSpec((1, C), lambda g: (0, 0)),
            pl.BlockSpec((1, C), lambda g: (0, 0)),
            pl.BlockSpec((1, C), lambda g: (0, 0)),
        ],
        out_specs=pl.BlockSpec((_BB, C, HW), lambda g: (g, 0, 0)),
        out_shape=jax.ShapeDtypeStruct((B, C, HW), ft.dtype),
        scratch_shapes=[pltpu.VMEM((B, C), jnp.float32)],
    )(taskvar, ft3, task_table, var_table,
      task_gamma.reshape(1, C), task_beta.reshape(1, C),
      var_gamma.reshape(1, C), var_beta.reshape(1, C))
    return out3.reshape(B, C, H, W)
